# Initial kernel scaffold; baseline (speedup 1.0000x reference)
import functools

import jax
import jax.numpy as jnp
from jax import lax
from jax.experimental import pallas as pl
from jax.experimental.pallas import tpu as pltpu
from jax.experimental.pallas import tpu_sc as plsc

B = 4096
L = 10
T = 20
N = B * L
D_AB = 16
D_T = 32
D_OUT = 64

NC = 2
NS = 16
NW = NC * NS
PER_TILE = N // NW
CH = 64
NCHUNK = PER_TILE // CH
SUB = 128
NSUB = CH * T // SUB
SCALE = 1.0 / T


def _sc_body(ida, idb, idt, e_id, e_c1, e_text, out,
             idxa_v, idxb_v, idxt_v, a_v, b_v, t_v, out_v, sem):
    wid = lax.axis_index("s") * NC + lax.axis_index("c")
    base = wid * PER_TILE

    def chunk_body(g, carry):
        b0 = base + g * CH
        t0 = (base * T) // SUB + g * NSUB

        pltpu.sync_copy(ida.at[pl.ds(b0, CH)], idxa_v)
        pltpu.sync_copy(idb.at[pl.ds(b0, CH)], idxb_v)
        pltpu.sync_copy(idt.at[pl.ds(t0, NSUB)], idxt_v)

        dmas = [
            pltpu.async_copy(e_id.at[idxa_v], a_v, sem),
            pltpu.async_copy(e_c1.at[idxb_v], b_v, sem),
        ]
        for j in range(NSUB):
            dmas.append(
                pltpu.async_copy(e_text.at[idxt_v.at[j]],
                                 t_v.at[pl.ds(j * SUB, SUB)], sem))
        for d in dmas:
            d.wait()

        def item_body(i, c2):
            r0 = i * T
            acc_a = t_v[r0, pl.ds(0, 16)]
            acc_b = t_v[r0, pl.ds(16, 16)]
            for j in range(1, T):
                acc_a = acc_a + t_v[r0 + j, pl.ds(0, 16)]
                acc_b = acc_b + t_v[r0 + j, pl.ds(16, 16)]
            out_v[i, pl.ds(0, 16)] = a_v[i, :]
            out_v[i, pl.ds(16, 16)] = b_v[i, :]
            out_v[i, pl.ds(32, 16)] = acc_a * SCALE
            out_v[i, pl.ds(48, 16)] = acc_b * SCALE
            return c2

        lax.fori_loop(0, CH, item_body, 0)

        pltpu.sync_copy(out_v, out.at[pl.ds(b0, CH)])
        return carry

    lax.fori_loop(0, NCHUNK, chunk_body, 0)


_sc_call = functools.partial(
    pl.kernel,
    mesh=plsc.VectorSubcoreMesh(core_axis_name="c", subcore_axis_name="s"),
    out_type=jax.ShapeDtypeStruct((N, D_OUT), jnp.float32),
    scratch_types=[
        pltpu.VMEM((CH,), jnp.int32),
        pltpu.VMEM((CH,), jnp.int32),
        pltpu.VMEM((NSUB, SUB), jnp.int32),
        pltpu.VMEM((CH, D_AB), jnp.float32),
        pltpu.VMEM((CH, D_AB), jnp.float32),
        pltpu.VMEM((CH * T, D_T), jnp.float32),
        pltpu.VMEM((CH, D_OUT), jnp.float32),
        pltpu.SemaphoreType.DMA,
    ],
)(_sc_body)


def kernel(item_id, contex_1, contex_2_tokens, E_id, E_c1, E_text):
    ida = item_id.reshape(N).astype(jnp.int32)
    idb = contex_1.reshape(N).astype(jnp.int32)
    idt = contex_2_tokens.reshape(N * T // SUB, SUB).astype(jnp.int32)
    out = _sc_call(ida, idb, idt, E_id, E_c1, E_text)
    return out.reshape(B, L, D_OUT)


# speedup vs baseline: 19.8303x; 19.8303x over previous
import functools

import jax
import jax.numpy as jnp
from jax import lax
from jax.experimental import pallas as pl
from jax.experimental.pallas import tpu as pltpu
from jax.experimental.pallas import tpu_sc as plsc

B = 4096
L = 10
T = 20
N = B * L
D_AB = 16
D_T = 32
D_OUT = 64

NC = 2
NS = 16
NW = NC * NS
PER_TILE = N // NW
CH = 64
NCHUNK = PER_TILE // CH
SUB = 128
NSUB = CH * T // SUB
SCALE = 1.0 / T


def _sc_body(ida, idb, idt, e_id, e_c1, e_text, out,
             idxa0, idxa1, idxb0, idxb1, idxt0, idxt1,
             av0, av1, bv0, bv1, tv0, tv1, ov0, ov1,
             si0, si1, sg0, sg1, so0, so1):
    idxa = [idxa0, idxa1]
    idxb = [idxb0, idxb1]
    idxt = [idxt0, idxt1]
    av = [av0, av1]
    bv = [bv0, bv1]
    tv = [tv0, tv1]
    ov = [ov0, ov1]
    si = [si0, si1]
    sg = [sg0, sg1]
    so = [so0, so1]

    wid = lax.axis_index("s") * NC + lax.axis_index("c")
    base = wid * PER_TILE

    def idx_copies(gg, s):
        n0 = base + gg * CH
        return [
            pltpu.make_async_copy(ida.at[pl.ds(n0, CH)], idxa[s], si[s]),
            pltpu.make_async_copy(idb.at[pl.ds(n0, CH)], idxb[s], si[s]),
            pltpu.make_async_copy(idt.at[pl.ds(n0 * T, CH * T)], idxt[s], si[s]),
        ]

    def gather_copies(s):
        cps = [
            pltpu.make_async_copy(e_id.at[idxa[s]], av[s], sg[s]),
            pltpu.make_async_copy(e_c1.at[idxb[s]], bv[s], sg[s]),
        ]
        for j in range(NSUB):
            cps.append(pltpu.make_async_copy(
                e_text.at[idxt[s].at[pl.ds(j * SUB, SUB)]],
                tv[s].at[pl.ds(j * SUB, SUB)], sg[s]))
        return cps

    def out_copy(gg, s):
        n0 = base + gg * CH
        return pltpu.make_async_copy(ov[s], out.at[pl.ds(n0, CH)], so[s])

    def fire_idx(gg, s):
        for c in idx_copies(gg, s):
            c.start()

    def wait_idx(s):
        for c in idx_copies(0, s):
            c.wait()

    def fire_gathers(s):
        for c in gather_copies(s):
            c.start()

    def wait_gathers(s):
        for c in gather_copies(s):
            c.wait()

    def compute(gg, s):
        a_v, b_v, t_v, o_v = av[s], bv[s], tv[s], ov[s]

        @plsc.parallel_loop(0, CH, unroll=2)
        def item_body(i):
            r0 = i * T
            acc_a = t_v[r0, pl.ds(0, 16)]
            acc_b = t_v[r0, pl.ds(16, 16)]
            for j in range(1, T):
                acc_a = acc_a + t_v[r0 + j, pl.ds(0, 16)]
                acc_b = acc_b + t_v[r0 + j, pl.ds(16, 16)]
            o_v[i, pl.ds(0, 16)] = a_v[i, :]
            o_v[i, pl.ds(16, 16)] = b_v[i, :]
            o_v[i, pl.ds(32, 16)] = acc_a * SCALE
            o_v[i, pl.ds(48, 16)] = acc_b * SCALE

        out_copy(gg, s).start()

    fire_idx(0, 0)
    fire_idx(1, 1)
    wait_idx(0)
    fire_gathers(0)

    def loop_body(gi, carry):
        for s in (0, 1):
            gg = 2 * gi + s
            wait_gathers(s)
            fire_idx(gg + 2, s)
            wait_idx(1 - s)
            fire_gathers(1 - s)

            @pl.when(gg >= 2)
            def _():
                out_copy(0, s).wait()

            compute(gg, s)
        return carry

    lax.fori_loop(0, (NCHUNK - 2) // 2, loop_body, 0)

    wait_gathers(0)
    wait_idx(1)
    fire_gathers(1)
    out_copy(0, 0).wait()
    compute(NCHUNK - 2, 0)
    wait_gathers(1)
    out_copy(0, 1).wait()
    compute(NCHUNK - 1, 1)
    out_copy(0, 0).wait()
    out_copy(0, 1).wait()


_sc_call = functools.partial(
    pl.kernel,
    mesh=plsc.VectorSubcoreMesh(core_axis_name="c", subcore_axis_name="s"),
    out_type=jax.ShapeDtypeStruct((N, D_OUT), jnp.float32),
    compiler_params=pltpu.CompilerParams(use_tc_tiling_on_sc=False),
    scratch_types=[
        pltpu.VMEM((CH,), jnp.int32),
        pltpu.VMEM((CH,), jnp.int32),
        pltpu.VMEM((CH,), jnp.int32),
        pltpu.VMEM((CH,), jnp.int32),
        pltpu.VMEM((CH * T,), jnp.int32),
        pltpu.VMEM((CH * T,), jnp.int32),
        pltpu.VMEM((CH, D_AB), jnp.float32),
        pltpu.VMEM((CH, D_AB), jnp.float32),
        pltpu.VMEM((CH, D_AB), jnp.float32),
        pltpu.VMEM((CH, D_AB), jnp.float32),
        pltpu.VMEM((CH * T, D_T), jnp.float32),
        pltpu.VMEM((CH * T, D_T), jnp.float32),
        pltpu.VMEM((CH, D_OUT), jnp.float32),
        pltpu.VMEM((CH, D_OUT), jnp.float32),
        pltpu.SemaphoreType.DMA,
        pltpu.SemaphoreType.DMA,
        pltpu.SemaphoreType.DMA,
        pltpu.SemaphoreType.DMA,
        pltpu.SemaphoreType.DMA,
        pltpu.SemaphoreType.DMA,
    ],
)(_sc_body)


def kernel(item_id, contex_1, contex_2_tokens, E_id, E_c1, E_text):
    ida = item_id.reshape(N).astype(jnp.int32)
    idb = contex_1.reshape(N).astype(jnp.int32)
    idt = contex_2_tokens.reshape(N * T).astype(jnp.int32)
    out = _sc_call(ida, idb, idt, E_id, E_c1, E_text)
    return out.reshape(B, L, D_OUT)


# speedup vs baseline: 25.8492x; 1.3035x over previous
import functools

import jax
import jax.numpy as jnp
from jax import lax
from jax.experimental import pallas as pl
from jax.experimental.pallas import tpu as pltpu
from jax.experimental.pallas import tpu_sc as plsc

B = 4096
L = 10
T = 20
N = B * L
D_AB = 16
D_T = 32
D_OUT = 64

NC = 2
NS = 16
NW = NC * NS
PER_TILE = N // NW
CH = 64
NCHUNK = PER_TILE // CH
SUB = 128
NSUB = CH * T // SUB
SCALE = 1.0 / T


def _sc_body(ida, idb, idt, e_id, e_c1, e_text, out,
             idxa0, idxa1, idxb0, idxb1, idxt0, idxt1,
             av0, av1, bv0, bv1, tv0, tv1, ov0, ov1,
             si0, si1, sg0, sg1, so0, so1):
    idxa = [idxa0, idxa1]
    idxb = [idxb0, idxb1]
    idxt = [idxt0, idxt1]
    av = [av0, av1]
    bv = [bv0, bv1]
    tv = [tv0, tv1]
    ov = [ov0, ov1]
    si = [si0, si1]
    sg = [sg0, sg1]
    so = [so0, so1]

    wid = lax.axis_index("s") * NC + lax.axis_index("c")
    base = wid * PER_TILE

    def idx_copies(gg, s):
        n0 = base + gg * CH
        l = n0 // B
        b0 = n0 - l * B
        return [
            pltpu.make_async_copy(ida.at[pl.ds(n0, CH)], idxa[s], si[s]),
            pltpu.make_async_copy(idb.at[pl.ds(n0, CH)], idxb[s], si[s]),
            pltpu.make_async_copy(idt.at[pl.ds(l * T, T), pl.ds(b0, CH)],
                                  idxt[s], si[s]),
        ]

    def gather_copies(s):
        cps = [
            pltpu.make_async_copy(e_id.at[idxa[s]], av[s], sg[s]),
            pltpu.make_async_copy(e_c1.at[idxb[s]], bv[s], sg[s]),
        ]
        for j in range(T):
            cps.append(pltpu.make_async_copy(
                e_text.at[idxt[s].at[j]],
                tv[s].at[pl.ds(j * CH, CH)], sg[s]))
        return cps

    def out_copy(gg, s):
        n0 = base + gg * CH
        return pltpu.make_async_copy(ov[s], out.at[pl.ds(n0, CH)], so[s])

    def fire_idx(gg, s):
        for c in idx_copies(gg, s):
            c.start()

    def wait_idx(s):
        for c in idx_copies(0, s):
            c.wait()

    def fire_gathers(s):
        for c in gather_copies(s):
            c.start()

    def wait_gathers(s):
        for c in gather_copies(s):
            c.wait()

    def compute(gg, s):
        a_v, b_v, t_v, o_v = av[s], bv[s], tv[s], ov[s]

        @plsc.parallel_loop(0, CH, unroll=2)
        def item_body(i):
            acc_a = t_v[i, pl.ds(0, 16)]
            acc_b = t_v[i, pl.ds(16, 16)]
            for j in range(1, T):
                acc_a = acc_a + t_v[j * CH + i, pl.ds(0, 16)]
                acc_b = acc_b + t_v[j * CH + i, pl.ds(16, 16)]
            o_v[i, pl.ds(0, 16)] = a_v[i, :]
            o_v[i, pl.ds(16, 16)] = b_v[i, :]
            o_v[i, pl.ds(32, 16)] = acc_a * SCALE
            o_v[i, pl.ds(48, 16)] = acc_b * SCALE

        out_copy(gg, s).start()

    fire_idx(0, 0)
    fire_idx(1, 1)
    wait_idx(0)
    fire_gathers(0)

    def loop_body(gi, carry):
        for s in (0, 1):
            gg = 2 * gi + s
            wait_gathers(s)
            fire_idx(gg + 2, s)
            wait_idx(1 - s)
            fire_gathers(1 - s)

            @pl.when(gg >= 2)
            def _():
                out_copy(0, s).wait()

            compute(gg, s)
        return carry

    lax.fori_loop(0, (NCHUNK - 2) // 2, loop_body, 0)

    wait_gathers(0)
    wait_idx(1)
    fire_gathers(1)
    out_copy(0, 0).wait()
    compute(NCHUNK - 2, 0)
    wait_gathers(1)
    out_copy(0, 1).wait()
    compute(NCHUNK - 1, 1)
    out_copy(0, 0).wait()
    out_copy(0, 1).wait()


_sc_call = functools.partial(
    pl.kernel,
    mesh=plsc.VectorSubcoreMesh(core_axis_name="c", subcore_axis_name="s"),
    out_type=jax.ShapeDtypeStruct((N, D_OUT), jnp.float32),
    compiler_params=pltpu.CompilerParams(use_tc_tiling_on_sc=False),
    scratch_types=[
        pltpu.VMEM((CH,), jnp.int32),
        pltpu.VMEM((CH,), jnp.int32),
        pltpu.VMEM((CH,), jnp.int32),
        pltpu.VMEM((CH,), jnp.int32),
        pltpu.VMEM((T, CH), jnp.int32),
        pltpu.VMEM((T, CH), jnp.int32),
        pltpu.VMEM((CH, D_AB), jnp.float32),
        pltpu.VMEM((CH, D_AB), jnp.float32),
        pltpu.VMEM((CH, D_AB), jnp.float32),
        pltpu.VMEM((CH, D_AB), jnp.float32),
        pltpu.VMEM((CH * T, D_T), jnp.float32),
        pltpu.VMEM((CH * T, D_T), jnp.float32),
        pltpu.VMEM((CH, D_OUT), jnp.float32),
        pltpu.VMEM((CH, D_OUT), jnp.float32),
        pltpu.SemaphoreType.DMA,
        pltpu.SemaphoreType.DMA,
        pltpu.SemaphoreType.DMA,
        pltpu.SemaphoreType.DMA,
        pltpu.SemaphoreType.DMA,
        pltpu.SemaphoreType.DMA,
    ],
)(_sc_body)


def kernel(item_id, contex_1, contex_2_tokens, E_id, E_c1, E_text):
    ida = jnp.transpose(item_id).reshape(N).astype(jnp.int32)
    idb = jnp.transpose(contex_1).reshape(N).astype(jnp.int32)
    idt = jnp.transpose(contex_2_tokens, (1, 2, 0)).reshape(L * T, B)
    idt = idt.astype(jnp.int32)
    out = _sc_call(ida, idb, idt, E_id, E_c1, E_text)
    return jnp.transpose(out.reshape(L, B, D_OUT), (1, 0, 2))


# speedup vs baseline: 26.8931x; 1.0404x over previous
import functools

import jax
import jax.numpy as jnp
from jax import lax
from jax.experimental import pallas as pl
from jax.experimental.pallas import tpu as pltpu
from jax.experimental.pallas import tpu_sc as plsc

B = 4096
L = 10
T = 20
N = B * L
D_AB = 16
D_T = 32
D_OUT = 64

NC = 2
NS = 16
NW = NC * NS
PER_TILE = N // NW
CH = 64
NCHUNK = PER_TILE // CH
SUB = 128
NSUB = CH * T // SUB
SCALE = 1.0 / T


def _sc_body(ida, idb, idt, e_id, e_c1, e_text, out,
             idxa0, idxa1, idxb0, idxb1, idxt0, idxt1,
             av0, av1, bv0, bv1, tv0, tv1, ov0, ov1,
             si0, si1, sg0, sg1, so0, so1):
    idxa = [idxa0, idxa1]
    idxb = [idxb0, idxb1]
    idxt = [idxt0, idxt1]
    av = [av0, av1]
    bv = [bv0, bv1]
    tv = [tv0, tv1]
    ov = [ov0, ov1]
    si = [si0, si1]
    sg = [sg0, sg1]
    so = [so0, so1]

    wid = lax.axis_index("s") * NC + lax.axis_index("c")
    base = wid * PER_TILE

    def idx_copies(gg, s):
        n0 = base + gg * CH
        l = n0 // B
        b0 = n0 - l * B
        return [
            pltpu.make_async_copy(ida.at[pl.ds(n0, CH)], idxa[s], si[s]),
            pltpu.make_async_copy(idb.at[pl.ds(n0, CH)], idxb[s], si[s]),
            pltpu.make_async_copy(idt.at[pl.ds(l * T, T), pl.ds(b0, CH)],
                                  idxt[s], si[s]),
        ]

    def gather_copies(s):
        cps = [
            pltpu.make_async_copy(e_id.at[idxa[s]], av[s], sg[s]),
            pltpu.make_async_copy(e_c1.at[idxb[s]], bv[s], sg[s]),
        ]
        for j in range(T):
            cps.append(pltpu.make_async_copy(
                e_text.at[idxt[s].at[j]],
                tv[s].at[pl.ds(j * CH, CH)], sg[s]))
        return cps

    def out_copies(gg, s):
        n0 = base + gg * CH
        return [
            pltpu.make_async_copy(av[s], out.at[pl.ds(n0, CH), pl.ds(0, 16)],
                                  so[s]),
            pltpu.make_async_copy(bv[s], out.at[pl.ds(n0, CH), pl.ds(16, 16)],
                                  so[s]),
            pltpu.make_async_copy(ov[s], out.at[pl.ds(n0, CH), pl.ds(32, 32)],
                                  so[s]),
        ]

    def wait_out(s):
        for c in out_copies(0, s):
            c.wait()

    def fire_idx(gg, s):
        for c in idx_copies(gg, s):
            c.start()

    def wait_idx(s):
        for c in idx_copies(0, s):
            c.wait()

    def fire_gathers(s):
        for c in gather_copies(s):
            c.start()

    def wait_gathers(s):
        for c in gather_copies(s):
            c.wait()

    def compute(gg, s):
        t_v, o_v = tv[s], ov[s]

        @plsc.parallel_loop(0, CH, unroll=2)
        def item_body(i):
            acc_a, acc_b = plsc.unpack(
                t_v[i, :], format=plsc.PackFormat.INTERLEAVED)
            for j in range(1, T):
                u_a, u_b = plsc.unpack(
                    t_v[j * CH + i, :], format=plsc.PackFormat.INTERLEAVED)
                acc_a = acc_a + u_a
                acc_b = acc_b + u_b
            o_v[i, pl.ds(0, 16)] = acc_a * SCALE
            o_v[i, pl.ds(16, 16)] = acc_b * SCALE

        for c in out_copies(gg, s):
            c.start()

    fire_idx(0, 0)
    fire_idx(1, 1)
    wait_idx(0)
    fire_gathers(0)

    def loop_body(gi, carry):
        for s in (0, 1):
            gg = 2 * gi + s
            wait_gathers(s)
            fire_idx(gg + 2, s)
            wait_idx(1 - s)

            @pl.when(gg >= 1)
            def _():
                wait_out(1 - s)

            fire_gathers(1 - s)
            compute(gg, s)
        return carry

    lax.fori_loop(0, (NCHUNK - 2) // 2, loop_body, 0)

    wait_gathers(0)
    wait_idx(1)
    wait_out(1)
    fire_gathers(1)
    compute(NCHUNK - 2, 0)
    wait_gathers(1)
    compute(NCHUNK - 1, 1)
    wait_out(0)
    wait_out(1)


_sc_call = functools.partial(
    pl.kernel,
    mesh=plsc.VectorSubcoreMesh(core_axis_name="c", subcore_axis_name="s"),
    out_type=jax.ShapeDtypeStruct((N, D_OUT), jnp.float32),
    compiler_params=pltpu.CompilerParams(use_tc_tiling_on_sc=False,
                                         needs_layout_passes=False),
    scratch_types=[
        pltpu.VMEM((CH,), jnp.int32),
        pltpu.VMEM((CH,), jnp.int32),
        pltpu.VMEM((CH,), jnp.int32),
        pltpu.VMEM((CH,), jnp.int32),
        pltpu.VMEM((T, CH), jnp.int32),
        pltpu.VMEM((T, CH), jnp.int32),
        pltpu.VMEM((CH, D_AB), jnp.float32),
        pltpu.VMEM((CH, D_AB), jnp.float32),
        pltpu.VMEM((CH, D_AB), jnp.float32),
        pltpu.VMEM((CH, D_AB), jnp.float32),
        pltpu.VMEM((CH * T, D_T), jnp.bfloat16),
        pltpu.VMEM((CH * T, D_T), jnp.bfloat16),
        pltpu.VMEM((CH, D_T), jnp.float32),
        pltpu.VMEM((CH, D_T), jnp.float32),
        pltpu.SemaphoreType.DMA,
        pltpu.SemaphoreType.DMA,
        pltpu.SemaphoreType.DMA,
        pltpu.SemaphoreType.DMA,
        pltpu.SemaphoreType.DMA,
        pltpu.SemaphoreType.DMA,
    ],
)(_sc_body)


def kernel(item_id, contex_1, contex_2_tokens, E_id, E_c1, E_text):
    ida = jnp.transpose(item_id).reshape(N).astype(jnp.int32)
    idb = jnp.transpose(contex_1).reshape(N).astype(jnp.int32)
    idt = jnp.transpose(contex_2_tokens, (1, 2, 0)).reshape(L * T, B)
    idt = idt.astype(jnp.int32)
    perm = []
    for k in range(D_T // 2):
        perm += [k, D_T // 2 + k]
    e_sw = E_text[:, jnp.array(perm, dtype=jnp.int32)].astype(jnp.bfloat16)
    out = _sc_call(ida, idb, idt, E_id, E_c1, e_sw)
    return jnp.transpose(out.reshape(L, B, D_OUT), (1, 0, 2))


# speedup vs baseline: 27.0065x; 1.0042x over previous
import functools

import jax
import jax.numpy as jnp
from jax import lax
from jax.experimental import pallas as pl
from jax.experimental.pallas import tpu as pltpu
from jax.experimental.pallas import tpu_sc as plsc

B = 4096
L = 10
T = 20
N = B * L
D_AB = 16
D_T = 32
D_OUT = 64

NC = 2
NS = 16
NW = NC * NS
PER_TILE = N // NW
CH = 64
NCHUNK = PER_TILE // CH
SUB = 128
NSUB = CH * T // SUB
SCALE = 1.0 / T


def _sc_body(ida, idb, idt, e_id, e_c1, e_text, out,
             idxa0, idxa1, idxb0, idxb1, idxt0, idxt1,
             av0, av1, bv0, bv1, tv0, tv1, ov0, ov1,
             si0, si1, sg0, sg1, so0, so1):
    idxa = [idxa0, idxa1]
    idxb = [idxb0, idxb1]
    idxt = [idxt0, idxt1]
    av = [av0, av1]
    bv = [bv0, bv1]
    tv = [tv0, tv1]
    ov = [ov0, ov1]
    si = [si0, si1]
    sg = [sg0, sg1]
    so = [so0, so1]

    wid = lax.axis_index("s") * NC + lax.axis_index("c")
    base = wid * PER_TILE

    def idx_copies(gg, s):
        n0 = base + gg * CH
        l = n0 // B
        b0 = n0 - l * B
        return [
            pltpu.make_async_copy(ida.at[pl.ds(n0, CH)], idxa[s], si[s]),
            pltpu.make_async_copy(idb.at[pl.ds(n0, CH)], idxb[s], si[s]),
            pltpu.make_async_copy(idt.at[pl.ds(l * T, T), pl.ds(b0, CH)],
                                  idxt[s], si[s]),
        ]

    def gather_copies(s):
        cps = [
            pltpu.make_async_copy(e_id.at[idxa[s]], av[s], sg[s]),
            pltpu.make_async_copy(e_c1.at[idxb[s]], bv[s], sg[s]),
        ]
        for j in range(T):
            cps.append(pltpu.make_async_copy(
                e_text.at[idxt[s].at[j]],
                tv[s].at[pl.ds(j * CH, CH)], sg[s]))
        return cps

    def out_copies(gg, s):
        n0 = base + gg * CH
        return [
            pltpu.make_async_copy(av[s], out.at[pl.ds(n0, CH), pl.ds(0, 16)],
                                  so[s]),
            pltpu.make_async_copy(bv[s], out.at[pl.ds(n0, CH), pl.ds(16, 16)],
                                  so[s]),
            pltpu.make_async_copy(ov[s], out.at[pl.ds(n0, CH), pl.ds(32, 32)],
                                  so[s]),
        ]

    def wait_out(s):
        for c in out_copies(0, s):
            c.wait()

    def fire_idx(gg, s):
        for c in idx_copies(gg, s):
            c.start()

    def wait_idx(s):
        for c in idx_copies(0, s):
            c.wait()

    def fire_gathers(s):
        for c in gather_copies(s):
            c.start()

    def wait_gathers(s):
        for c in gather_copies(s):
            c.wait()

    def compute(gg, s):
        t_v, o_v = tv[s], ov[s]

        @plsc.parallel_loop(0, CH, unroll=2)
        def item_body(i):
            acc = t_v[i, :]
            for j in range(1, T):
                acc = acc + t_v[j * CH + i, :]
            acc_a, acc_b = plsc.unpack(
                acc, format=plsc.PackFormat.INTERLEAVED)
            o_v[i, pl.ds(0, 16)] = acc_a * SCALE
            o_v[i, pl.ds(16, 16)] = acc_b * SCALE

        for c in out_copies(gg, s):
            c.start()

    fire_idx(0, 0)
    fire_idx(1, 1)
    wait_idx(0)
    fire_gathers(0)

    def loop_body(gi, carry):
        for s in (0, 1):
            gg = 2 * gi + s
            wait_gathers(s)
            fire_idx(gg + 2, s)
            wait_idx(1 - s)

            @pl.when(gg >= 1)
            def _():
                wait_out(1 - s)

            fire_gathers(1 - s)
            compute(gg, s)
        return carry

    lax.fori_loop(0, (NCHUNK - 2) // 2, loop_body, 0)

    wait_gathers(0)
    wait_idx(1)
    wait_out(1)
    fire_gathers(1)
    compute(NCHUNK - 2, 0)
    wait_gathers(1)
    compute(NCHUNK - 1, 1)
    wait_out(0)
    wait_out(1)


_sc_call = functools.partial(
    pl.kernel,
    mesh=plsc.VectorSubcoreMesh(core_axis_name="c", subcore_axis_name="s"),
    out_type=jax.ShapeDtypeStruct((N, D_OUT), jnp.float32),
    compiler_params=pltpu.CompilerParams(use_tc_tiling_on_sc=False,
                                         needs_layout_passes=False),
    scratch_types=[
        pltpu.VMEM((CH,), jnp.int32),
        pltpu.VMEM((CH,), jnp.int32),
        pltpu.VMEM((CH,), jnp.int32),
        pltpu.VMEM((CH,), jnp.int32),
        pltpu.VMEM((T, CH), jnp.int32),
        pltpu.VMEM((T, CH), jnp.int32),
        pltpu.VMEM((CH, D_AB), jnp.float32),
        pltpu.VMEM((CH, D_AB), jnp.float32),
        pltpu.VMEM((CH, D_AB), jnp.float32),
        pltpu.VMEM((CH, D_AB), jnp.float32),
        pltpu.VMEM((CH * T, D_T), jnp.bfloat16),
        pltpu.VMEM((CH * T, D_T), jnp.bfloat16),
        pltpu.VMEM((CH, D_T), jnp.float32),
        pltpu.VMEM((CH, D_T), jnp.float32),
        pltpu.SemaphoreType.DMA,
        pltpu.SemaphoreType.DMA,
        pltpu.SemaphoreType.DMA,
        pltpu.SemaphoreType.DMA,
        pltpu.SemaphoreType.DMA,
        pltpu.SemaphoreType.DMA,
    ],
)(_sc_body)


def kernel(item_id, contex_1, contex_2_tokens, E_id, E_c1, E_text):
    ida = jnp.transpose(item_id).reshape(N).astype(jnp.int32)
    idb = jnp.transpose(contex_1).reshape(N).astype(jnp.int32)
    idt = jnp.transpose(contex_2_tokens, (1, 2, 0)).reshape(L * T, B)
    idt = idt.astype(jnp.int32)
    perm = []
    for k in range(D_T // 2):
        perm += [k, D_T // 2 + k]
    e_sw = E_text[:, jnp.array(perm, dtype=jnp.int32)].astype(jnp.bfloat16)
    out = _sc_call(ida, idb, idt, E_id, E_c1, e_sw)
    return jnp.transpose(out.reshape(L, B, D_OUT), (1, 0, 2))


# speedup vs baseline: 28.2479x; 1.0460x over previous
import functools

import jax
import jax.numpy as jnp
from jax import lax
from jax.experimental import pallas as pl
from jax.experimental.pallas import tpu as pltpu
from jax.experimental.pallas import tpu_sc as plsc

B = 4096
L = 10
T = 20
N = B * L
D_AB = 16
D_T = 32
D_OUT = 64

NC = 2
NS = 16
NW = NC * NS
PER_TILE = N // NW
CH = 128
NCHUNK = PER_TILE // CH
SUB = 128
NSUB = CH * T // SUB
SCALE = 1.0 / T


def _sc_body(ida, idb, idt, e_id, e_c1, e_text, out,
             idxa0, idxa1, idxb0, idxb1, idxt0, idxt1,
             av0, av1, bv0, bv1, tv0, tv1, ov0, ov1,
             si0, si1, sg0, sg1, so0, so1):
    idxa = [idxa0, idxa1]
    idxb = [idxb0, idxb1]
    idxt = [idxt0, idxt1]
    av = [av0, av1]
    bv = [bv0, bv1]
    tv = [tv0, tv1]
    ov = [ov0, ov1]
    si = [si0, si1]
    sg = [sg0, sg1]
    so = [so0, so1]

    wid = lax.axis_index("s") * NC + lax.axis_index("c")
    base = wid * PER_TILE

    def idx_copies(gg, s):
        n0 = base + gg * CH
        l = n0 // B
        b0 = n0 - l * B
        return [
            pltpu.make_async_copy(ida.at[pl.ds(n0, CH)], idxa[s], si[s]),
            pltpu.make_async_copy(idb.at[pl.ds(n0, CH)], idxb[s], si[s]),
            pltpu.make_async_copy(idt.at[pl.ds(l * T, T), pl.ds(b0, CH)],
                                  idxt[s], si[s]),
        ]

    def gather_copies(s):
        cps = [
            pltpu.make_async_copy(e_id.at[idxa[s]], av[s], sg[s]),
            pltpu.make_async_copy(e_c1.at[idxb[s]], bv[s], sg[s]),
        ]
        for j in range(T):
            cps.append(pltpu.make_async_copy(
                e_text.at[idxt[s].at[j]],
                tv[s].at[pl.ds(j * CH, CH)], sg[s]))
        return cps

    def out_copies(gg, s):
        n0 = base + gg * CH
        return [
            pltpu.make_async_copy(av[s], out.at[pl.ds(n0, CH), pl.ds(0, 16)],
                                  so[s]),
            pltpu.make_async_copy(bv[s], out.at[pl.ds(n0, CH), pl.ds(16, 16)],
                                  so[s]),
            pltpu.make_async_copy(ov[s], out.at[pl.ds(n0, CH), pl.ds(32, 32)],
                                  so[s]),
        ]

    def wait_out(s):
        for c in out_copies(0, s):
            c.wait()

    def fire_idx(gg, s):
        for c in idx_copies(gg, s):
            c.start()

    def wait_idx(s):
        for c in idx_copies(0, s):
            c.wait()

    def fire_gathers(s):
        for c in gather_copies(s):
            c.start()

    def wait_gathers(s):
        for c in gather_copies(s):
            c.wait()

    def compute(gg, s):
        t_v, o_v = tv[s], ov[s]

        @plsc.parallel_loop(0, CH, unroll=2)
        def item_body(i):
            acc = t_v[i, :]
            for j in range(1, T):
                acc = acc + t_v[j * CH + i, :]
            acc_a, acc_b = plsc.unpack(
                acc, format=plsc.PackFormat.INTERLEAVED)
            o_v[i, pl.ds(0, 16)] = acc_a * SCALE
            o_v[i, pl.ds(16, 16)] = acc_b * SCALE

        for c in out_copies(gg, s):
            c.start()

    fire_idx(0, 0)
    fire_idx(1, 1)
    wait_idx(0)
    fire_gathers(0)

    def loop_body(gi, carry):
        for s in (0, 1):
            gg = 2 * gi + s
            wait_gathers(s)
            fire_idx(gg + 2, s)
            wait_idx(1 - s)

            @pl.when(gg >= 1)
            def _():
                wait_out(1 - s)

            fire_gathers(1 - s)
            compute(gg, s)
        return carry

    lax.fori_loop(0, (NCHUNK - 2) // 2, loop_body, 0)

    wait_gathers(0)
    wait_idx(1)
    wait_out(1)
    fire_gathers(1)
    compute(NCHUNK - 2, 0)
    wait_gathers(1)
    compute(NCHUNK - 1, 1)
    wait_out(0)
    wait_out(1)


_sc_call = functools.partial(
    pl.kernel,
    mesh=plsc.VectorSubcoreMesh(core_axis_name="c", subcore_axis_name="s"),
    out_type=jax.ShapeDtypeStruct((N, D_OUT), jnp.float32),
    compiler_params=pltpu.CompilerParams(use_tc_tiling_on_sc=False,
                                         needs_layout_passes=False),
    scratch_types=[
        pltpu.VMEM((CH,), jnp.int32),
        pltpu.VMEM((CH,), jnp.int32),
        pltpu.VMEM((CH,), jnp.int32),
        pltpu.VMEM((CH,), jnp.int32),
        pltpu.VMEM((T, CH), jnp.int32),
        pltpu.VMEM((T, CH), jnp.int32),
        pltpu.VMEM((CH, D_AB), jnp.float32),
        pltpu.VMEM((CH, D_AB), jnp.float32),
        pltpu.VMEM((CH, D_AB), jnp.float32),
        pltpu.VMEM((CH, D_AB), jnp.float32),
        pltpu.VMEM((CH * T, D_T), jnp.bfloat16),
        pltpu.VMEM((CH * T, D_T), jnp.bfloat16),
        pltpu.VMEM((CH, D_T), jnp.float32),
        pltpu.VMEM((CH, D_T), jnp.float32),
        pltpu.SemaphoreType.DMA,
        pltpu.SemaphoreType.DMA,
        pltpu.SemaphoreType.DMA,
        pltpu.SemaphoreType.DMA,
        pltpu.SemaphoreType.DMA,
        pltpu.SemaphoreType.DMA,
    ],
)(_sc_body)


def kernel(item_id, contex_1, contex_2_tokens, E_id, E_c1, E_text):
    ida = jnp.transpose(item_id).reshape(N).astype(jnp.int32)
    idb = jnp.transpose(contex_1).reshape(N).astype(jnp.int32)
    idt = jnp.transpose(contex_2_tokens, (1, 2, 0)).reshape(L * T, B)
    idt = idt.astype(jnp.int32)
    perm = []
    for k in range(D_T // 2):
        perm += [k, D_T // 2 + k]
    e_sw = E_text[:, jnp.array(perm, dtype=jnp.int32)].astype(jnp.bfloat16)
    out = _sc_call(ida, idb, idt, E_id, E_c1, e_sw)
    return jnp.transpose(out.reshape(L, B, D_OUT), (1, 0, 2))
